# Initial kernel scaffold; baseline (speedup 1.0000x reference)
import jax
import jax.numpy as jnp
from jax.experimental import pallas as pl


def kernel(x, edge_index, edge_attr, fc1_w, fc1_b, fc2_w, fc2_b, g0, mu0, sigma0, root0, b0, g1, mu1, sigma1, root1, b1):
    raise NotImplementedError("write your pallas kernel here")


# speedup vs baseline: 1.7828x; 1.7828x over previous
import functools

import jax
import jax.numpy as jnp
from jax import lax
from jax.experimental import pallas as pl
from jax.experimental.pallas import tpu as pltpu
from jax.experimental.pallas import tpu_sc as plsc

NUM_GRAPHS = 100
INPUT_SIZE = 100
HID = 128
BOTTLENECK = 256
FFN = 1024
K = 5
N = NUM_GRAPHS * INPUT_SIZE
E = 320000
EPS = 1e-15

_PREC = jax.lax.Precision.HIGHEST

_NC = 2
_NS = 16
_NW = _NC * _NS
_EB = 40
_EPW = E // _NW
_NBATCH = _EPW // _EB
_EBP = 48
_RPT = 624
_RTAIL = N - _NS * _RPT


def _elu(v):
    return jnp.where(v > 0, v, jnp.exp(jnp.minimum(v, 0.0)) - 1.0)


_MLP_BLK = 2560


def _mlp_body(x_ref, w1_ref, b1_ref, w2_ref, b2_ref, out_ref, mid_ref):
    @pl.when(pl.program_id(0) == 0)
    def _():
        mid_ref[...] = _elu(
            jnp.dot(x_ref[...], w1_ref[...],
                    preferred_element_type=jnp.float32, precision=_PREC)
            + b1_ref[...])

    out_ref[...] = _elu(
        jnp.dot(mid_ref[...], w2_ref[...],
                preferred_element_type=jnp.float32, precision=_PREC)
        + b2_ref[...])


def _mlp(x, w1, b1, w2, b2):
    nb = (INPUT_SIZE * HID) // _MLP_BLK
    return pl.pallas_call(
        _mlp_body,
        grid=(nb,),
        in_specs=[
            pl.BlockSpec((NUM_GRAPHS, BOTTLENECK), lambda i: (0, 0)),
            pl.BlockSpec((BOTTLENECK, FFN), lambda i: (0, 0)),
            pl.BlockSpec((1, FFN), lambda i: (0, 0)),
            pl.BlockSpec((FFN, _MLP_BLK), lambda i: (0, i)),
            pl.BlockSpec((1, _MLP_BLK), lambda i: (0, i)),
        ],
        out_specs=pl.BlockSpec((NUM_GRAPHS, _MLP_BLK), lambda i: (0, i)),
        out_shape=jax.ShapeDtypeStruct((NUM_GRAPHS, INPUT_SIZE * HID),
                                       jnp.float32),
        scratch_shapes=[pltpu.VMEM((NUM_GRAPHS, FFN), jnp.float32)],
    )(x, w1, b1, w2, b2)


_ROWB = 2000


def _xg_body(xin_ref, g_ref, root_ref, xg_ref, r_ref):
    xg_ref[...] = jnp.dot(xin_ref[...], g_ref[...],
                          preferred_element_type=jnp.float32, precision=_PREC)
    r_ref[...] = jnp.dot(xin_ref[...], root_ref[...],
                         preferred_element_type=jnp.float32, precision=_PREC)


def _xg(xin, g, root):
    return pl.pallas_call(
        _xg_body,
        grid=(N // _ROWB,),
        in_specs=[
            pl.BlockSpec((_ROWB, HID), lambda i: (i, 0)),
            pl.BlockSpec((HID, K * HID), lambda i: (0, 0)),
            pl.BlockSpec((HID, HID), lambda i: (0, 0)),
        ],
        out_specs=[
            pl.BlockSpec((_ROWB, K * HID), lambda i: (i, 0)),
            pl.BlockSpec((_ROWB, HID), lambda i: (i, 0)),
        ],
        out_shape=[
            jax.ShapeDtypeStruct((N, K * HID), jnp.float32),
            jax.ShapeDtypeStruct((N, HID), jnp.float32),
        ],
    )(xin, g, root)


_HCHUNK = 6400
_HROWS = 80


def _hist_body(dstc_ref, out_ref, acc_ref):
    i = pl.program_id(0)

    @pl.when(i == 0)
    def _():
        acc_ref[...] = jnp.zeros_like(acc_ref)

    dc = dstc_ref[...]
    a_ids = lax.broadcasted_iota(jnp.int32, (_HCHUNK, _HROWS), 1)
    b_ids = lax.broadcasted_iota(jnp.int32, (_HCHUNK, HID), 1)
    mask_at = ((dc // HID) == a_ids).astype(jnp.float32)
    onehot_b = ((dc % HID) == b_ids).astype(jnp.float32)
    acc_ref[...] += lax.dot_general(
        mask_at, onehot_b, dimension_numbers=(((0,), (0,)), ((), ())),
        preferred_element_type=jnp.float32)

    @pl.when(i == pl.num_programs(0) - 1)
    def _():
        out_ref[...] = acc_ref[...]


def _hist(dst):
    nchunk = E // _HCHUNK
    dstc = dst.reshape(E, 1)
    return pl.pallas_call(
        _hist_body,
        grid=(nchunk,),
        in_specs=[pl.BlockSpec((_HCHUNK, 1), lambda i: (i, 0))],
        out_specs=pl.BlockSpec((_HROWS, HID), lambda i: (0, 0)),
        out_shape=jax.ShapeDtypeStruct((_HROWS, HID), jnp.float32),
        scratch_shapes=[pltpu.VMEM((_HROWS, HID), jnp.float32)],
    )(dstc)


@functools.cache
def _make_conv():
    mesh = plsc.VectorSubcoreMesh(core_axis_name="c", subcore_axis_name="s")
    out_type = jax.ShapeDtypeStruct((_NC, N, HID), jnp.float32)

    scratch = [
        pltpu.VMEM_SHARED((N, HID), jnp.float32),
        pltpu.VMEM((_EB, K * HID), jnp.float32),
        pltpu.VMEM((_EB, HID), jnp.float32),
        pltpu.VMEM((K * _EBP + 16,), jnp.float32),
        pltpu.VMEM((_EBP,), jnp.float32),
        pltpu.VMEM((_EB,), jnp.int32),
        pltpu.VMEM((_EB,), jnp.int32),
        pltpu.VMEM((HID,), jnp.float32),
        pltpu.SemaphoreType.DMA,
    ]

    def body(xg_hbm, src_hbm, dst_hbm, attr_hbm, prm_hbm, agg_out,
             agg_sp, rows_v, msg_v, gw_v, attr_v, sidx_v, didx_v, prm_v,
             sem):
        c = lax.axis_index("c")
        s = lax.axis_index("s")
        wid = s * _NC + c

        zero16 = jnp.zeros((16,), jnp.float32)

        pltpu.sync_copy(prm_hbm, prm_v)
        for q in range(_EBP // 16):
            attr_v[pl.ds(q * 16, 16)] = jnp.zeros((16,), jnp.float32)

        def fill_zero(i, carry):
            for cb in range(HID // 16):
                msg_v[i, pl.ds(cb * 16, 16)] = zero16
            return carry

        lax.fori_loop(0, _EB, fill_zero, 0)

        rbase = s * _RPT
        nfull = _RPT // _EB
        tail = _RPT % _EB
        for t in range(nfull):
            pltpu.sync_copy(msg_v, agg_sp.at[pl.ds(rbase + t * _EB, _EB)])
        if tail:
            pltpu.sync_copy(msg_v.at[pl.ds(0, tail)],
                            agg_sp.at[pl.ds(rbase + nfull * _EB, tail)])

        @pl.when(s == _NS - 1)
        def _():
            pltpu.sync_copy(msg_v.at[pl.ds(0, _RTAIL)],
                            agg_sp.at[pl.ds(_NS * _RPT, _RTAIL)])

        plsc.subcore_barrier()

        ew0 = wid * _EPW
        pvec = prm_v[pl.ds(0, 16)]

        def batch(j, carry):
            eb = ew0 + j * _EB
            pltpu.sync_copy(src_hbm.at[pl.ds(eb, _EB)], sidx_v)
            pltpu.sync_copy(dst_hbm.at[pl.ds(eb, _EB)], didx_v)
            pltpu.sync_copy(attr_hbm.at[pl.ds(eb, _EB)],
                            attr_v.at[pl.ds(0, _EB)])
            pltpu.async_copy(xg_hbm.at[sidx_v], rows_v, sem).wait()

            for v in range(_EBP // 16):
                a = attr_v[pl.ds(v * 16, 16)]
                for k in range(K):
                    t = a - pvec[k]
                    gw_v[pl.ds(k * _EBP + v * 16, 16)] = jnp.exp(
                        t * t * pvec[8 + k])

            def ebody(i, ecarry):
                w0 = gw_v[pl.ds(i, 16)][0]
                w1 = gw_v[pl.ds(_EBP + i, 16)][0]
                w2 = gw_v[pl.ds(2 * _EBP + i, 16)][0]
                w3 = gw_v[pl.ds(3 * _EBP + i, 16)][0]
                w4 = gw_v[pl.ds(4 * _EBP + i, 16)][0]
                for cb in range(HID // 16):
                    o = cb * 16
                    acc = rows_v[i, pl.ds(o, 16)] * w0
                    acc = acc + rows_v[i, pl.ds(HID + o, 16)] * w1
                    acc = acc + rows_v[i, pl.ds(2 * HID + o, 16)] * w2
                    acc = acc + rows_v[i, pl.ds(3 * HID + o, 16)] * w3
                    acc = acc + rows_v[i, pl.ds(4 * HID + o, 16)] * w4
                    msg_v[i, pl.ds(o, 16)] = acc
                return ecarry

            lax.fori_loop(0, _EB, ebody, 0)

            pltpu.sync_copy(msg_v, agg_sp.at[didx_v], add=True)
            return carry

        lax.fori_loop(0, _NBATCH, batch, 0)

        plsc.subcore_barrier()
        pltpu.sync_copy(agg_sp.at[pl.ds(rbase, _RPT)],
                        agg_out.at[c, pl.ds(rbase, _RPT)])

        @pl.when(s == _NS - 1)
        def _():
            pltpu.sync_copy(agg_sp.at[pl.ds(_NS * _RPT, _RTAIL)],
                            agg_out.at[c, pl.ds(_NS * _RPT, _RTAIL)])

    return pl.kernel(body, mesh=mesh, out_type=out_type,
                     scratch_types=scratch)


_EROWB = 2000


def _epi_body(apply_act, aggp_ref, cnt_ref, r_ref, b_ref, h_ref, out_ref):
    agg = aggp_ref[0] + aggp_ref[1]
    val = (agg / jnp.clip(cnt_ref[...], 1.0, None)
           + r_ref[...] + b_ref[...])
    if apply_act:
        val = _elu(val)
    out_ref[...] = val + h_ref[...]


def _epilogue(aggp, cnt, r, b2d, h, apply_act):
    body = functools.partial(_epi_body, apply_act)
    return pl.pallas_call(
        body,
        grid=(N // _EROWB,),
        in_specs=[
            pl.BlockSpec((_NC, _EROWB, HID), lambda i: (0, i, 0)),
            pl.BlockSpec((_EROWB, 1), lambda i: (i, 0)),
            pl.BlockSpec((_EROWB, HID), lambda i: (i, 0)),
            pl.BlockSpec((1, HID), lambda i: (0, 0)),
            pl.BlockSpec((_EROWB, HID), lambda i: (i, 0)),
        ],
        out_specs=pl.BlockSpec((_EROWB, HID), lambda i: (i, 0)),
        out_shape=jax.ShapeDtypeStruct((N, HID), jnp.float32),
    )(aggp, cnt, r, b2d, h)


def kernel(x, edge_index, edge_attr, fc1_w, fc1_b, fc2_w, fc2_b,
           g0, mu0, sigma0, root0, b0, g1, mu1, sigma1, root1, b1):
    src = edge_index[0]
    dst = edge_index[1]

    def pack_params(mu, sigma):
        alpha = -0.5 / (EPS + sigma[:, 0] ** 2)
        p = jnp.zeros((HID,), jnp.float32)
        return p.at[0:K].set(mu[:, 0]).at[8:8 + K].set(alpha)

    p0 = pack_params(mu0, sigma0)
    p1 = pack_params(mu1, sigma1)

    h = _mlp(x, fc1_w, fc1_b.reshape(1, FFN),
             fc2_w, fc2_b.reshape(1, INPUT_SIZE * HID)).reshape(N, HID)

    cnt = _hist(dst).reshape(_HROWS * HID)[:N].reshape(N, 1)

    xg0, r0 = _xg(h, g0, root0)
    aggp0 = _make_conv()(xg0, src, dst, edge_attr, p0)
    out0 = _epilogue(aggp0, cnt, r0, b0.reshape(1, HID), h, True)

    xg1, r1 = _xg(out0, g1, root1)
    aggp1 = _make_conv()(xg1, src, dst, edge_attr, p1)
    out = _epilogue(aggp1, cnt, r1, b1.reshape(1, HID), h, False)
    return out
